# baseline (speedup 1.0000x reference)
import functools

import jax
import jax.numpy as jnp
from jax import lax
from jax.experimental import pallas as pl
from jax.experimental.pallas import tpu as pltpu
from jax.experimental.pallas import tpu_sc as plsc

N_VOCAB = 100000
EMBED_DIM = 64
BATCH = 4096
HIST = 50

NC = 2
NS = 16
NW = NC * NS
BT = BATCH // NW

_mesh = plsc.VectorSubcoreMesh(core_axis_name="c", subcore_axis_name="s")


@functools.partial(
    pl.kernel,
    mesh=_mesh,
    out_type=jax.ShapeDtypeStruct((HIST, 8, NW, 8, 128), jnp.float32),
    scratch_types=[
        pltpu.VMEM((HIST, BT), jnp.int32),
        pltpu.VMEM((HIST, BT), jnp.int32),
        pltpu.VMEM((2, BT, 128), jnp.float32),
        pltpu.VMEM((2, 8, 8, 133), jnp.float32),
        [pltpu.SemaphoreType.DMA] * 2,
        [pltpu.SemaphoreType.DMA] * 2,
    ],
    compiler_params=pltpu.CompilerParams(
        use_tc_tiling_on_sc=True, needs_layout_passes=False),
)
def _embed_lookup(xt_hbm, table_hbm, out_hbm, idx_v, idx2_v, rows_v, tile_v,
                  gsems, ssems):
    wid = lax.axis_index("s") * NC + lax.axis_index("c")
    pltpu.sync_copy(xt_hbm.at[:, pl.ds(wid * BT, BT)], idx_v)

    @plsc.parallel_loop(0, HIST * (BT // 16), step=1, unroll=8)
    def _(i):
        h = i >> 3
        c = i & 7
        idx2_v[h, pl.ds(16 * c, 16)] = idx_v[h, pl.ds(16 * c, 16)] >> 1

    iota16 = lax.iota(jnp.int32, 16)
    tevs = [(iota16 + 16 * c) >> 3 for c in range(4)]
    eevs = [(iota16 + 16 * c) & 7 for c in range(4)]

    def fire(h, b):
        pltpu.async_copy(table_hbm.at[idx2_v.at[h]], rows_v.at[b], gsems[b])

    def wait_gather(h, b):
        pltpu.make_async_copy(
            table_hbm.at[idx2_v.at[h]], rows_v.at[b], gsems[b]).wait()

    def transpose(h, b):
        @plsc.parallel_loop(0, BT // 16, step=1, unroll=2)
        def _(r0q):
            r0 = r0q * 16
            halfs = (idx_v[h, pl.ds(r0, 16)] & 1) * EMBED_DIM
            for l in range(16):
                half = halfs[l]
                colb = jnp.full((16,), 0, jnp.int32) + (r0 + l)
                for c in range(4):
                    vals = rows_v[b, r0 + l, pl.ds(half + 16 * c, 16)]
                    plsc.store_scatter(
                        tile_v.at[b], [tevs[c], eevs[c], colb], vals)

    def start_store(h, b):
        pltpu.async_copy(
            tile_v.at[b, :, :, pl.ds(0, 128)], out_hbm.at[h, :, wid],
            ssems[b])

    def wait_store(h, b):
        pltpu.make_async_copy(
            tile_v.at[b, :, :, pl.ds(0, 128)], out_hbm.at[h, :, wid],
            ssems[b]).wait()

    fire(0, 0)
    fire(1, 1)

    def body(p, carry):
        for b in (0, 1):
            h = 2 * p + b
            wait_gather(h, b)

            @pl.when(p >= 1)
            def _():
                wait_store(h, b)

            transpose(h, b)
            start_store(h, b)
            fire(h + 2, b)
        return carry

    lax.fori_loop(0, HIST // 2 - 1, body, 0)

    for b in (0, 1):
        h = HIST - 2 + b
        wait_gather(h, b)
        wait_store(h, b)
        transpose(h, b)
        start_store(h, b)
    for b in (0, 1):
        wait_store(HIST - 2 + b, b)


def kernel(x, weight):
    xt = x.T.astype(jnp.int32)
    t2 = weight.reshape(N_VOCAB // 2, 128)
    out5 = _embed_lookup(xt, t2)
    return out5.transpose(2, 4, 0, 1, 3).reshape(BATCH, HIST, EMBED_DIM)


# speedup vs baseline: 2.2001x; 2.2001x over previous
import functools

import jax
import jax.numpy as jnp
from jax import lax
from jax.experimental import pallas as pl
from jax.experimental.pallas import tpu as pltpu
from jax.experimental.pallas import tpu_sc as plsc

N_VOCAB = 100000
EMBED_DIM = 64
BATCH = 4096
HIST = 50

NC = 2
NS = 16
NW = NC * NS
BT = BATCH // NW

_mesh = plsc.VectorSubcoreMesh(core_axis_name="c", subcore_axis_name="s")


@functools.partial(
    pl.kernel,
    mesh=_mesh,
    out_type=jax.ShapeDtypeStruct((HIST, 8, NW, 8, 128), jnp.float32),
    scratch_types=[
        pltpu.VMEM((HIST, BT), jnp.int32),
        pltpu.VMEM((2, BT, EMBED_DIM), jnp.float32),
        pltpu.VMEM((2, 8, 8, 133), jnp.float32),
        [pltpu.SemaphoreType.DMA] * 2,
        [pltpu.SemaphoreType.DMA] * 2,
    ],
    compiler_params=pltpu.CompilerParams(
        use_tc_tiling_on_sc=False, needs_layout_passes=False),
)
def _embed_lookup(xt_hbm, table_hbm, out_hbm, idx_v, rows_v, tile_v,
                  gsems, ssems):
    wid = lax.axis_index("s") * NC + lax.axis_index("c")
    pltpu.sync_copy(xt_hbm.at[:, pl.ds(wid * BT, BT)], idx_v)

    iota16 = lax.iota(jnp.int32, 16)
    tevs = [(iota16 + 16 * c) >> 3 for c in range(4)]
    eevs = [(iota16 + 16 * c) & 7 for c in range(4)]

    def fire(h, b):
        pltpu.async_copy(table_hbm.at[idx_v.at[h]], rows_v.at[b], gsems[b])

    def wait_gather(h, b):
        pltpu.make_async_copy(
            table_hbm.at[idx_v.at[h]], rows_v.at[b], gsems[b]).wait()

    def transpose(b):
        @plsc.parallel_loop(0, BT, step=1, unroll=8)
        def _(bp):
            colb = jnp.full((16,), 0, jnp.int32) + bp
            for c in range(4):
                vals = rows_v[b, bp, pl.ds(16 * c, 16)]
                plsc.store_scatter(
                    tile_v.at[b], [tevs[c], eevs[c], colb], vals)

    def start_store(h, b):
        pltpu.async_copy(
            tile_v.at[b, :, :, pl.ds(0, 128)], out_hbm.at[h, :, wid],
            ssems[b])

    def wait_store(h, b):
        pltpu.make_async_copy(
            tile_v.at[b, :, :, pl.ds(0, 128)], out_hbm.at[h, :, wid],
            ssems[b]).wait()

    fire(0, 0)
    fire(1, 1)

    def body(p, carry):
        for b in (0, 1):
            h = 2 * p + b
            wait_gather(h, b)

            @pl.when(p >= 1)
            def _():
                wait_store(h, b)

            transpose(b)
            start_store(h, b)
            fire(h + 2, b)
        return carry

    lax.fori_loop(0, HIST // 2 - 1, body, 0)

    for b in (0, 1):
        h = HIST - 2 + b
        wait_gather(h, b)
        wait_store(h, b)
        transpose(b)
        start_store(h, b)
    for b in (0, 1):
        wait_store(HIST - 2 + b, b)


def kernel(x, weight):
    xt = x.T.astype(jnp.int32)
    out5 = _embed_lookup(xt, weight)
    return out5.transpose(2, 4, 0, 1, 3).reshape(BATCH, HIST, EMBED_DIM)


# speedup vs baseline: 2.2001x; 1.0000x over previous
import functools

import jax
import jax.numpy as jnp
from jax import lax
from jax.experimental import pallas as pl
from jax.experimental.pallas import tpu as pltpu
from jax.experimental.pallas import tpu_sc as plsc

N_VOCAB = 100000
EMBED_DIM = 64
BATCH = 4096
HIST = 50

NC = 2
NS = 16
NW = NC * NS
BT = BATCH // NW

_mesh = plsc.VectorSubcoreMesh(core_axis_name="c", subcore_axis_name="s")


@functools.partial(
    pl.kernel,
    mesh=_mesh,
    out_type=jax.ShapeDtypeStruct((HIST, 8, NW, 8, 128), jnp.float32),
    scratch_types=[
        pltpu.VMEM((HIST, BT), jnp.int32),
        pltpu.VMEM((2, BT, EMBED_DIM), jnp.float32),
        pltpu.VMEM((2, 8, 8, 133), jnp.float32),
        [pltpu.SemaphoreType.DMA] * 2,
        [pltpu.SemaphoreType.DMA] * 2,
    ],
    compiler_params=pltpu.CompilerParams(
        use_tc_tiling_on_sc=False, needs_layout_passes=False),
)
def _embed_lookup(xt_hbm, table_hbm, out_hbm, idx_v, rows_v, tile_v,
                  gsems, ssems):
    wid = lax.axis_index("s") * NC + lax.axis_index("c")
    pltpu.sync_copy(xt_hbm.at[:, pl.ds(wid * BT, BT)], idx_v)

    iota16 = lax.iota(jnp.int32, 16)
    tevs = [(iota16 + 16 * c) >> 3 for c in range(4)]
    eevs = [(iota16 + 16 * c) & 7 for c in range(4)]

    def fire(h, b):
        pltpu.async_copy(table_hbm.at[idx_v.at[h]], rows_v.at[b], gsems[b])

    def wait_gather(h, b):
        pltpu.make_async_copy(
            table_hbm.at[idx_v.at[h]], rows_v.at[b], gsems[b]).wait()

    def transpose(b):
        @plsc.parallel_loop(0, BT, step=1, unroll=4)
        def _(bp):
            colb = jnp.full((16,), 0, jnp.int32) + bp
            for c in range(4):
                vals = rows_v[b, bp, pl.ds(16 * c, 16)]
                plsc.store_scatter(
                    tile_v.at[b], [tevs[c], eevs[c], colb], vals)

    def start_store(h, b):
        pltpu.async_copy(
            tile_v.at[b, :, :, pl.ds(0, 128)], out_hbm.at[h, :, wid],
            ssems[b])

    def wait_store(h, b):
        pltpu.make_async_copy(
            tile_v.at[b, :, :, pl.ds(0, 128)], out_hbm.at[h, :, wid],
            ssems[b]).wait()

    fire(0, 0)
    fire(1, 1)

    def body(p, carry):
        for b in (0, 1):
            h = 2 * p + b
            wait_gather(h, b)

            @pl.when(p >= 1)
            def _():
                wait_store(h, b)

            transpose(b)
            start_store(h, b)
            fire(h + 2, b)
        return carry

    lax.fori_loop(0, HIST // 2 - 1, body, 0)

    for b in (0, 1):
        h = HIST - 2 + b
        wait_gather(h, b)
        wait_store(h, b)
        transpose(b)
        start_store(h, b)
    for b in (0, 1):
        wait_store(HIST - 2 + b, b)


def kernel(x, weight):
    xt = x.T.astype(jnp.int32)
    out5 = _embed_lookup(xt, weight)
    return out5.transpose(2, 4, 0, 1, 3).reshape(BATCH, HIST, EMBED_DIM)
